# baseline (speedup 1.0000x reference)
import functools

import jax
import jax.numpy as jnp
from jax import lax
from jax.experimental import pallas as pl
from jax.experimental.pallas import tpu as pltpu
from jax.experimental.pallas import tpu_sc as plsc

N = 10000
E = 320000
N_PAD = 10240
C = 128
CH = 160
E_PAD = 16 * CH * C
RB = 256
NBLK = N_PAD // RB
ROWS_PER_TILE = N_PAD // 16
W1H = 80
W2H = 64

_f32 = jnp.float32


def _mesh():
    return plsc.VectorSubcoreMesh(core_axis_name="c", subcore_axis_name="s")


def _sc_params():
    return pltpu.CompilerParams(use_tc_tiling_on_sc=False)


def _fill_const(ref, rows, d, val):
    nv = d // 16

    def row(i, _):
        for j in range(nv):
            ref[i, pl.ds(j * 16, 16)] = jnp.full((16,), val, _f32)
        return 0

    lax.fori_loop(0, rows, row, 0)


def _sc_degree(src3):
    @functools.partial(
        pl.kernel,
        mesh=_mesh(),
        compiler_params=_sc_params(),
        out_type=jax.ShapeDtypeStruct((2, N_PAD, 16), _f32),
        scratch_types=[
            pltpu.VMEM((CH, C), jnp.int32),
            pltpu.VMEM((C, 16), _f32),
            pltpu.VMEM((C, 16), _f32),
            pltpu.VMEM_SHARED((N_PAD, 16), _f32),
        ],
    )
    def k(src_hbm, out_hbm, idx_v, ones_v, zer_v, acc):
        cid = lax.axis_index("c")
        sid = lax.axis_index("s")
        _fill_const(ones_v, C, 16, 1.0)
        _fill_const(zer_v, C, 16, 0.0)
        base = sid * ROWS_PER_TILE
        nfull, rem = divmod(ROWS_PER_TILE, C)
        for j in range(nfull):
            pltpu.sync_copy(zer_v, acc.at[pl.ds(base + j * C, C)])
        if rem:
            pltpu.sync_copy(zer_v.at[pl.ds(0, rem)],
                            acc.at[pl.ds(base + nfull * C, rem)])
        plsc.subcore_barrier()

        pltpu.sync_copy(src_hbm.at[sid], idx_v)
        lo = cid * (CH // 2)
        hi = lo + (CH // 2)

        def body(g, _):
            pltpu.sync_copy(ones_v, acc.at[idx_v.at[g]], add=True)
            return 0

        lax.fori_loop(lo, hi, body, 0)
        plsc.subcore_barrier()
        pltpu.sync_copy(acc.at[pl.ds(base, ROWS_PER_TILE)],
                        out_hbm.at[cid, pl.ds(base, ROWS_PER_TILE)])

    return k(src3)


def _sc_prop(u2, src3, dst3, w):
    @functools.partial(
        pl.kernel,
        mesh=_mesh(),
        compiler_params=_sc_params(),
        out_type=jax.ShapeDtypeStruct((2, N_PAD, w), _f32),
        scratch_types=[
            pltpu.VMEM((CH, C), jnp.int32),
            pltpu.VMEM((CH, C), jnp.int32),
            pltpu.VMEM((C, w), _f32),
            pltpu.VMEM((C, w), _f32),
            pltpu.VMEM_SHARED((N_PAD, w), _f32),
            pltpu.SemaphoreType.DMA,
            pltpu.SemaphoreType.DMA,
        ],
    )
    def k(u_hbm, src_hbm, dst_hbm, out_hbm,
          src_v, dst_v, g0, g1, acc, sem0, sem1):
        cid = lax.axis_index("c")
        sid = lax.axis_index("s")
        base = sid * ROWS_PER_TILE

        _fill_const(g0, C, w, 0.0)
        nfull, rem = divmod(ROWS_PER_TILE, C)
        for j in range(nfull):
            pltpu.sync_copy(g0, acc.at[pl.ds(base + j * C, C)])
        if rem:
            pltpu.sync_copy(g0.at[pl.ds(0, rem)],
                            acc.at[pl.ds(base + nfull * C, rem)])
        plsc.subcore_barrier()

        pltpu.sync_copy(src_hbm.at[sid], src_v)
        pltpu.sync_copy(dst_hbm.at[sid], dst_v)
        tab = u_hbm.at[cid]

        def body(g, _):
            e = g * 2
            cpa = pltpu.async_copy(tab.at[src_v.at[e]], g0, sem0)
            cpb = pltpu.async_copy(tab.at[src_v.at[e + 1]], g1, sem1)
            cpa.wait()
            pltpu.sync_copy(g0, acc.at[dst_v.at[e]], add=True)
            cpb.wait()
            pltpu.sync_copy(g1, acc.at[dst_v.at[e + 1]], add=True)
            return 0

        lax.fori_loop(0, CH // 2, body, 0)
        plsc.subcore_barrier()
        pltpu.sync_copy(acc.at[pl.ds(base, ROWS_PER_TILE)],
                        out_hbm.at[cid, pl.ds(base, ROWS_PER_TILE)])

    return k(u2, src3, dst3)


def _row_spec(d):
    return pl.BlockSpec((RB, d), lambda i: (i, 0))


def _split_spec(w):
    return pl.BlockSpec((2, RB, w), lambda i: (0, i, 0))


def _full_spec(shape):
    nd = len(shape)
    return pl.BlockSpec(shape, lambda i, _n=nd: (0,) * _n)


def _tc_prep(xp, pp, hist):
    def body(x_ref, p_ref, h_ref, u_ref, dis2_ref, sdeg_ref):
        deg = h_ref[0, :, 0:1] + h_ref[1, :, 0:1] + 1.0
        dis = lax.rsqrt(deg)
        u_ref[0, :, :] = x_ref[:, 0:80] * dis
        u_ref[1, :, 0:48] = x_ref[:, 80:128] * dis
        u_ref[1, :, 48:64] = p_ref[...] * dis
        u_ref[1, :, 64:80] = jnp.zeros((RB, 16), _f32)
        dis2_ref[...] = 1.0 / deg
        sdeg_ref[...] = jnp.sqrt(deg)

    return pl.pallas_call(
        body,
        grid=(NBLK,),
        in_specs=[
            _row_spec(128),
            _row_spec(16),
            pl.BlockSpec((2, RB, 16), lambda i: (0, i, 0)),
        ],
        out_specs=[_split_spec(W1H), _row_spec(1), _row_spec(1)],
        out_shape=[
            jax.ShapeDtypeStruct((2, N_PAD, W1H), _f32),
            jax.ShapeDtypeStruct((N_PAD, 1), _f32),
            jax.ShapeDtypeStruct((N_PAD, 1), _f32),
        ],
    )(xp, pp, hist)


def _tc_combine(y2, ukm1, ukm2, dis2, alpha, w):
    has_prev = ukm2 is not None

    def body(*refs):
        if has_prev:
            y_ref, a_ref, b_ref, s_ref, o_ref = refs
        else:
            y_ref, a_ref, s_ref, o_ref = refs
        s = s_ref[...]
        for c in (0, 1):
            o = (-alpha) * s * (y_ref[c] + a_ref[c])
            if has_prev:
                o = o - b_ref[c]
            o_ref[c, :, :] = o

    in_specs = [_split_spec(w), _split_spec(w)]
    args = [y2, ukm1]
    if has_prev:
        in_specs.append(_split_spec(w))
        args.append(ukm2)
    in_specs.append(_row_spec(1))
    args.append(dis2)
    return pl.pallas_call(
        body,
        grid=(NBLK,),
        in_specs=in_specs,
        out_specs=_split_spec(w),
        out_shape=jax.ShapeDtypeStruct((2, N_PAD, w), _f32),
    )(*args)


def _tc_layer1_final(us, sdeg, dis2, wc, b):
    def body(u0, u1, u2, u3, s_ref, s2_ref, w_ref, b_ref, o_ref):
        s = s_ref[...]
        acc = b_ref[...] + jnp.zeros((RB, 128), _f32)
        for k, u in enumerate((u0, u1, u2, u3)):
            acc += jnp.dot(u[0] * s, w_ref[pl.ds(144 * k, 80)],
                           preferred_element_type=_f32)
            acc += jnp.dot(u[1, :, 0:64] * s, w_ref[pl.ds(144 * k + 80, 64)],
                           preferred_element_type=_f32)
        h = jnp.maximum(acc, 0.0)
        dis = jnp.sqrt(s2_ref[...])
        o_ref[0, :, :] = h[:, 0:64] * dis
        o_ref[1, :, :] = h[:, 64:128] * dis

    return pl.pallas_call(
        body,
        grid=(NBLK,),
        in_specs=[_split_spec(W1H)] * 4
        + [_row_spec(1), _row_spec(1), _full_spec((576, 128)),
           _full_spec((1, 128))],
        out_specs=_split_spec(W2H),
        out_shape=jax.ShapeDtypeStruct((2, N_PAD, W2H), _f32),
    )(*us, sdeg, dis2, wc, b)


def _tc_layer2_final(us, sdeg, wc, b, wmu, bmu, wlv, blv):
    def body(u0, u1, u2, u3, s_ref, w_ref, b_ref,
             wmu_ref, bmu_ref, wlv_ref, blv_ref, mu_ref, lv_ref):
        s = s_ref[...]
        acc = b_ref[...] + jnp.zeros((RB, 128), _f32)
        for k, u in enumerate((u0, u1, u2, u3)):
            acc += jnp.dot(u[0] * s, w_ref[pl.ds(128 * k, 64)],
                           preferred_element_type=_f32)
            acc += jnp.dot(u[1] * s, w_ref[pl.ds(128 * k + 64, 64)],
                           preferred_element_type=_f32)
        h = jnp.maximum(acc, 0.0)
        mu_ref[...] = jnp.dot(h, wmu_ref[...],
                              preferred_element_type=_f32) + bmu_ref[...]
        lv_ref[...] = jnp.dot(h, wlv_ref[...],
                              preferred_element_type=_f32) + blv_ref[...]

    return pl.pallas_call(
        body,
        grid=(NBLK,),
        in_specs=[_split_spec(W2H)] * 4
        + [_row_spec(1), _full_spec((512, 128)), _full_spec((1, 128)),
           _full_spec((128, 64)), _full_spec((1, 64)),
           _full_spec((128, 64)), _full_spec((1, 64))],
        out_specs=[_row_spec(64), _row_spec(64)],
        out_shape=[
            jax.ShapeDtypeStruct((N_PAD, 64), _f32),
            jax.ShapeDtypeStruct((N_PAD, 64), _f32),
        ],
    )(*us, sdeg, wc, b, wmu, bmu, wlv, blv)


def kernel(x, edge_index, lap_pe, W1, b1, W2, b2, Wmu, bmu, Wlv, blv):
    pad_idx = jnp.full((E_PAD - E,), N, jnp.int32)
    src3 = jnp.concatenate([edge_index[0], pad_idx]).reshape(16, CH, C)
    dst3 = jnp.concatenate([edge_index[1], pad_idx]).reshape(16, CH, C)
    xp = jnp.pad(x, ((0, N_PAD - N), (0, 0)))
    pp = jnp.pad(lap_pe, ((0, N_PAD - N), (0, 0)))

    hist = _sc_degree(src3)
    u0, dis2, sdeg = _tc_prep(xp, pp, hist)

    y = _sc_prop(u0, src3, dst3, W1H)
    u1 = _tc_combine(y, u0, None, dis2, 1.0, W1H)
    y = _sc_prop(u1, src3, dst3, W1H)
    u2 = _tc_combine(y, u1, u0, dis2, 2.0, W1H)
    y = _sc_prop(u2, src3, dst3, W1H)
    u3 = _tc_combine(y, u2, u1, dis2, 2.0, W1H)

    v0 = _tc_layer1_final((u0, u1, u2, u3), sdeg, dis2,
                          W1.reshape(4 * 144, 128), b1.reshape(1, 128))

    y = _sc_prop(v0, src3, dst3, W2H)
    v1 = _tc_combine(y, v0, None, dis2, 1.0, W2H)
    y = _sc_prop(v1, src3, dst3, W2H)
    v2 = _tc_combine(y, v1, v0, dis2, 2.0, W2H)
    y = _sc_prop(v2, src3, dst3, W2H)
    v3 = _tc_combine(y, v2, v1, dis2, 2.0, W2H)

    mu, lv = _tc_layer2_final((v0, v1, v2, v3), sdeg,
                              W2.reshape(4 * 128, 128), b2.reshape(1, 128),
                              Wmu, bmu.reshape(1, 64),
                              Wlv, blv.reshape(1, 64))
    return mu[:N], lv[:N]


# speedup vs baseline: 1.9250x; 1.9250x over previous
import functools

import jax
import jax.numpy as jnp
from jax import lax
from jax.experimental import pallas as pl
from jax.experimental.pallas import tpu as pltpu
from jax.experimental.pallas import tpu_sc as plsc

N = 10000
E = 320000
N_PAD = 10240
C = 128
CH = 160
E_PAD = 16 * CH * C
RB = 256
NBLK = N_PAD // RB
ROWS_PER_TILE = N_PAD // 16
W1H = 80
W2H = 64

_f32 = jnp.float32


def _mesh():
    return plsc.VectorSubcoreMesh(core_axis_name="c", subcore_axis_name="s")


def _sc_params():
    return pltpu.CompilerParams(use_tc_tiling_on_sc=False)


def _fill_const(ref, rows, d, val):
    nv = d // 16

    def row(i, _):
        for j in range(nv):
            ref[i, pl.ds(j * 16, 16)] = jnp.full((16,), val, _f32)
        return 0

    lax.fori_loop(0, rows, row, 0)


def _sc_degree(src3):
    @functools.partial(
        pl.kernel,
        mesh=_mesh(),
        compiler_params=_sc_params(),
        out_type=jax.ShapeDtypeStruct((2, N_PAD, 16), _f32),
        scratch_types=[
            pltpu.VMEM((CH, C), jnp.int32),
            pltpu.VMEM((C, 16), _f32),
            pltpu.VMEM((C, 16), _f32),
            pltpu.VMEM_SHARED((N_PAD, 16), _f32),
        ],
    )
    def k(src_hbm, out_hbm, idx_v, ones_v, zer_v, acc):
        cid = lax.axis_index("c")
        sid = lax.axis_index("s")
        _fill_const(ones_v, C, 16, 1.0)
        _fill_const(zer_v, C, 16, 0.0)
        base = sid * ROWS_PER_TILE
        nfull, rem = divmod(ROWS_PER_TILE, C)
        for j in range(nfull):
            pltpu.sync_copy(zer_v, acc.at[pl.ds(base + j * C, C)])
        if rem:
            pltpu.sync_copy(zer_v.at[pl.ds(0, rem)],
                            acc.at[pl.ds(base + nfull * C, rem)])
        plsc.subcore_barrier()

        pltpu.sync_copy(src_hbm.at[sid], idx_v)
        lo = cid * (CH // 2)
        hi = lo + (CH // 2)

        def body(g, _):
            pltpu.sync_copy(ones_v, acc.at[idx_v.at[g]], add=True)
            return 0

        lax.fori_loop(lo, hi, body, 0)
        plsc.subcore_barrier()
        pltpu.sync_copy(acc.at[pl.ds(base, ROWS_PER_TILE)],
                        out_hbm.at[cid, pl.ds(base, ROWS_PER_TILE)])

    return k(src3)


def _sc_prop(u2, src3, dst3, w):
    @functools.partial(
        pl.kernel,
        mesh=_mesh(),
        compiler_params=_sc_params(),
        out_type=jax.ShapeDtypeStruct((2, N_PAD, w), _f32),
        scratch_types=[
            pltpu.VMEM((CH, C), jnp.int32),
            pltpu.VMEM((CH, C), jnp.int32),
            pltpu.VMEM((C, w), _f32),
            pltpu.VMEM((C, w), _f32),
            pltpu.VMEM_SHARED((N_PAD, w), _f32),
            pltpu.SemaphoreType.DMA,
            pltpu.SemaphoreType.DMA,
        ],
    )
    def k(u_hbm, src_hbm, dst_hbm, out_hbm,
          src_v, dst_v, g0, g1, acc, sem0, sem1):
        cid = lax.axis_index("c")
        sid = lax.axis_index("s")
        base = sid * ROWS_PER_TILE

        _fill_const(g0, C, w, 0.0)
        nfull, rem = divmod(ROWS_PER_TILE, C)
        for j in range(nfull):
            pltpu.sync_copy(g0, acc.at[pl.ds(base + j * C, C)])
        if rem:
            pltpu.sync_copy(g0.at[pl.ds(0, rem)],
                            acc.at[pl.ds(base + nfull * C, rem)])
        plsc.subcore_barrier()

        pltpu.sync_copy(src_hbm.at[sid], src_v)
        pltpu.sync_copy(dst_hbm.at[sid], dst_v)
        tab = u_hbm.at[cid]

        def body(g, _):
            e = g * 2
            cpa = pltpu.async_copy(tab.at[src_v.at[e]], g0, sem0)
            cpb = pltpu.async_copy(tab.at[src_v.at[e + 1]], g1, sem1)
            cpa.wait()
            pltpu.sync_copy(g0, acc.at[dst_v.at[e]], add=True)
            cpb.wait()
            pltpu.sync_copy(g1, acc.at[dst_v.at[e + 1]], add=True)
            return 0

        lax.fori_loop(0, CH // 2, body, 0)
        plsc.subcore_barrier()
        pltpu.sync_copy(acc.at[pl.ds(base, ROWS_PER_TILE)],
                        out_hbm.at[cid, pl.ds(base, ROWS_PER_TILE)])

    return k(u2, src3, dst3)


def _row_spec(d):
    return pl.BlockSpec((RB, d), lambda i: (i, 0))


def _split_spec(w):
    return pl.BlockSpec((2, RB, w), lambda i: (0, i, 0))


def _full_spec(shape):
    nd = len(shape)
    return pl.BlockSpec(shape, lambda i, _n=nd: (0,) * _n)


def _tc_prep(xp, pp, hist):
    def body(x_ref, p_ref, h_ref, u_ref, dis2_ref, sdeg_ref):
        deg = h_ref[0, :, 0:1] + h_ref[1, :, 0:1] + 1.0
        dis = lax.rsqrt(deg)
        u_ref[0, :, :] = x_ref[:, 0:80] * dis
        u_ref[1, :, 0:48] = x_ref[:, 80:128] * dis
        u_ref[1, :, 48:64] = p_ref[...] * dis
        u_ref[1, :, 64:80] = jnp.zeros((RB, 16), _f32)
        dis2_ref[...] = 1.0 / deg
        sdeg_ref[...] = jnp.sqrt(deg)

    return pl.pallas_call(
        body,
        grid=(NBLK,),
        in_specs=[
            _row_spec(128),
            _row_spec(16),
            pl.BlockSpec((2, RB, 16), lambda i: (0, i, 0)),
        ],
        out_specs=[_split_spec(W1H), _row_spec(1), _row_spec(1)],
        out_shape=[
            jax.ShapeDtypeStruct((2, N_PAD, W1H), _f32),
            jax.ShapeDtypeStruct((N_PAD, 1), _f32),
            jax.ShapeDtypeStruct((N_PAD, 1), _f32),
        ],
    )(xp, pp, hist)


def _tc_combine(y2, ukm1, ukm2, dis2, alpha, w):
    has_prev = ukm2 is not None

    def body(*refs):
        if has_prev:
            y_ref, a_ref, b_ref, s_ref, o_ref = refs
        else:
            y_ref, a_ref, s_ref, o_ref = refs
        s = s_ref[...]
        for c in (0, 1):
            o = (-alpha) * s * (y_ref[c] + a_ref[c])
            if has_prev:
                o = o - b_ref[c]
            o_ref[c, :, :] = o

    in_specs = [_split_spec(w), _split_spec(w)]
    args = [y2, ukm1]
    if has_prev:
        in_specs.append(_split_spec(w))
        args.append(ukm2)
    in_specs.append(_row_spec(1))
    args.append(dis2)
    return pl.pallas_call(
        body,
        grid=(NBLK,),
        in_specs=in_specs,
        out_specs=_split_spec(w),
        out_shape=jax.ShapeDtypeStruct((2, N_PAD, w), _f32),
    )(*args)


def _tc_layer1_final(us, sdeg, dis2, wc, b):
    def body(u0, u1, u2, u3, s_ref, s2_ref, w_ref, b_ref, o_ref):
        s = s_ref[...]
        acc = b_ref[...] + jnp.zeros((RB, 128), _f32)
        for k, u in enumerate((u0, u1, u2, u3)):
            acc += jnp.dot(u[0] * s, w_ref[pl.ds(144 * k, 80)],
                           preferred_element_type=_f32)
            acc += jnp.dot(u[1, :, 0:64] * s, w_ref[pl.ds(144 * k + 80, 64)],
                           preferred_element_type=_f32)
        h = jnp.maximum(acc, 0.0)
        dis = jnp.sqrt(s2_ref[...])
        o_ref[0, :, :] = h[:, 0:64] * dis
        o_ref[1, :, :] = h[:, 64:128] * dis

    return pl.pallas_call(
        body,
        grid=(NBLK,),
        in_specs=[_split_spec(W1H)] * 4
        + [_row_spec(1), _row_spec(1), _full_spec((576, 128)),
           _full_spec((1, 128))],
        out_specs=_split_spec(W2H),
        out_shape=jax.ShapeDtypeStruct((2, N_PAD, W2H), _f32),
    )(*us, sdeg, dis2, wc, b)


def _tc_layer2_final(us, sdeg, wc, b, wmu, bmu, wlv, blv):
    def body(u0, u1, u2, u3, s_ref, w_ref, b_ref,
             wmu_ref, bmu_ref, wlv_ref, blv_ref, mu_ref, lv_ref):
        s = s_ref[...]
        acc = b_ref[...] + jnp.zeros((RB, 128), _f32)
        for k, u in enumerate((u0, u1, u2, u3)):
            acc += jnp.dot(u[0] * s, w_ref[pl.ds(128 * k, 64)],
                           preferred_element_type=_f32)
            acc += jnp.dot(u[1] * s, w_ref[pl.ds(128 * k + 64, 64)],
                           preferred_element_type=_f32)
        h = jnp.maximum(acc, 0.0)
        mu_ref[...] = jnp.dot(h, wmu_ref[...],
                              preferred_element_type=_f32) + bmu_ref[...]
        lv_ref[...] = jnp.dot(h, wlv_ref[...],
                              preferred_element_type=_f32) + blv_ref[...]

    return pl.pallas_call(
        body,
        grid=(NBLK,),
        in_specs=[_split_spec(W2H)] * 4
        + [_row_spec(1), _full_spec((512, 128)), _full_spec((1, 128)),
           _full_spec((128, 64)), _full_spec((1, 64)),
           _full_spec((128, 64)), _full_spec((1, 64))],
        out_specs=[_row_spec(64), _row_spec(64)],
        out_shape=[
            jax.ShapeDtypeStruct((N_PAD, 64), _f32),
            jax.ShapeDtypeStruct((N_PAD, 64), _f32),
        ],
    )(*us, sdeg, wc, b, wmu, bmu, wlv, blv)


def kernel(x, edge_index, lap_pe, W1, b1, W2, b2, Wmu, bmu, Wlv, blv):
    pad_idx = N + jnp.arange(E_PAD - E, dtype=jnp.int32) % (N_PAD - N)
    src3 = jnp.concatenate([edge_index[0], pad_idx]).reshape(16, CH, C)
    dst3 = jnp.concatenate([edge_index[1], pad_idx]).reshape(16, CH, C)
    xp = jnp.pad(x, ((0, N_PAD - N), (0, 0)))
    pp = jnp.pad(lap_pe, ((0, N_PAD - N), (0, 0)))

    hist = _sc_degree(src3)
    u0, dis2, sdeg = _tc_prep(xp, pp, hist)

    y = _sc_prop(u0, src3, dst3, W1H)
    u1 = _tc_combine(y, u0, None, dis2, 1.0, W1H)
    y = _sc_prop(u1, src3, dst3, W1H)
    u2 = _tc_combine(y, u1, u0, dis2, 2.0, W1H)
    y = _sc_prop(u2, src3, dst3, W1H)
    u3 = _tc_combine(y, u2, u1, dis2, 2.0, W1H)

    v0 = _tc_layer1_final((u0, u1, u2, u3), sdeg, dis2,
                          W1.reshape(4 * 144, 128), b1.reshape(1, 128))

    y = _sc_prop(v0, src3, dst3, W2H)
    v1 = _tc_combine(y, v0, None, dis2, 1.0, W2H)
    y = _sc_prop(v1, src3, dst3, W2H)
    v2 = _tc_combine(y, v1, v0, dis2, 2.0, W2H)
    y = _sc_prop(v2, src3, dst3, W2H)
    v3 = _tc_combine(y, v2, v1, dis2, 2.0, W2H)

    mu, lv = _tc_layer2_final((v0, v1, v2, v3), sdeg,
                              W2.reshape(4 * 128, 128), b2.reshape(1, 128),
                              Wmu, bmu.reshape(1, 64),
                              Wlv, blv.reshape(1, 64))
    return mu[:N], lv[:N]


# speedup vs baseline: 2.4193x; 1.2568x over previous
import functools

import jax
import jax.numpy as jnp
from jax import lax
from jax.experimental import pallas as pl
from jax.experimental.pallas import tpu as pltpu
from jax.experimental.pallas import tpu_sc as plsc

N = 10000
E = 320000
N_PAD = 10240
C = 112
CH = 180
E_PAD = 16 * CH * C
RB = 256
NBLK = N_PAD // RB
ROWS_PER_TILE = N_PAD // 16
W1H = 80
W2H = 64

_f32 = jnp.float32


def _mesh():
    return plsc.VectorSubcoreMesh(core_axis_name="c", subcore_axis_name="s")


def _sc_params():
    return pltpu.CompilerParams(use_tc_tiling_on_sc=False)


def _fill_const(ref, rows, d, val):
    nv = d // 16

    def row(i, _):
        for j in range(nv):
            ref[i, pl.ds(j * 16, 16)] = jnp.full((16,), val, _f32)
        return 0

    lax.fori_loop(0, rows, row, 0)


def _sc_degree(src3):
    @functools.partial(
        pl.kernel,
        mesh=_mesh(),
        compiler_params=_sc_params(),
        out_type=jax.ShapeDtypeStruct((2, N_PAD, 16), _f32),
        scratch_types=[
            pltpu.VMEM((CH, C), jnp.int32),
            pltpu.VMEM((C, 16), _f32),
            pltpu.VMEM((C, 16), _f32),
            pltpu.VMEM_SHARED((N_PAD, 16), _f32),
        ],
    )
    def k(src_hbm, out_hbm, idx_v, ones_v, zer_v, acc):
        cid = lax.axis_index("c")
        sid = lax.axis_index("s")
        _fill_const(ones_v, C, 16, 1.0)
        _fill_const(zer_v, C, 16, 0.0)
        base = sid * ROWS_PER_TILE
        nfull, rem = divmod(ROWS_PER_TILE, C)
        for j in range(nfull):
            pltpu.sync_copy(zer_v, acc.at[pl.ds(base + j * C, C)])
        if rem:
            pltpu.sync_copy(zer_v.at[pl.ds(0, rem)],
                            acc.at[pl.ds(base + nfull * C, rem)])
        plsc.subcore_barrier()

        pltpu.sync_copy(src_hbm.at[sid], idx_v)
        lo = cid * (CH // 2)
        hi = lo + (CH // 2)

        def body(g, _):
            pltpu.sync_copy(ones_v, acc.at[idx_v.at[g]], add=True)
            return 0

        lax.fori_loop(lo, hi, body, 0)
        plsc.subcore_barrier()
        pltpu.sync_copy(acc.at[pl.ds(base, ROWS_PER_TILE)],
                        out_hbm.at[cid, pl.ds(base, ROWS_PER_TILE)])

    return k(src3)


def _sc_prop(u2, src3, dst3, w):
    @functools.partial(
        pl.kernel,
        mesh=_mesh(),
        compiler_params=_sc_params(),
        out_type=jax.ShapeDtypeStruct((2, N_PAD, w), _f32),
        scratch_types=[
            pltpu.VMEM((CH, C), jnp.int32),
            pltpu.VMEM((CH, C), jnp.int32),
            pltpu.VMEM((C, w), _f32),
            pltpu.VMEM((C, w), _f32),
            pltpu.VMEM((C, w), _f32),
            pltpu.VMEM((C, w), _f32),
            pltpu.VMEM_SHARED((N_PAD, w), _f32),
            pltpu.SemaphoreType.DMA,
            pltpu.SemaphoreType.DMA,
            pltpu.SemaphoreType.DMA,
            pltpu.SemaphoreType.DMA,
        ],
    )
    def k(u_hbm, src_hbm, dst_hbm, out_hbm,
          src_v, dst_v, b0, b1, b2, b3, acc, sg0, sg1, ss0, ss1):
        cid = lax.axis_index("c")
        sid = lax.axis_index("s")
        base = sid * ROWS_PER_TILE
        bufs = (b0, b1, b2, b3)
        sgs = (sg0, sg1)
        sss = (ss0, ss1)

        _fill_const(b0, C, w, 0.0)
        nfull, rem = divmod(ROWS_PER_TILE, C)
        for j in range(nfull):
            pltpu.sync_copy(b0, acc.at[pl.ds(base + j * C, C)])
        if rem:
            pltpu.sync_copy(b0.at[pl.ds(0, rem)],
                            acc.at[pl.ds(base + nfull * C, rem)])
        plsc.subcore_barrier()

        pltpu.sync_copy(src_hbm.at[sid], src_v)
        pltpu.sync_copy(dst_hbm.at[sid], dst_v)
        tab = u_hbm.at[cid]

        def wait_gather(slot):
            pltpu.make_async_copy(tab.at[pl.ds(0, C)], bufs[slot],
                                  sgs[slot % 2]).wait()

        def wait_scatter(slot):
            pltpu.make_async_copy(bufs[slot],
                                  out_hbm.at[cid, pl.ds(0, C)],
                                  sss[slot % 2]).wait()

        def step(g, slot, first, last):
            wait_gather(slot)
            pltpu.async_copy(bufs[slot], acc.at[dst_v.at[g]],
                             sss[slot % 2], add=True)
            if not first:
                wait_scatter((slot + 2) % 4)
            if not last:
                pltpu.async_copy(tab.at[src_v.at[g + 2]],
                                 bufs[(slot + 2) % 4], sgs[slot % 2])

        pltpu.async_copy(tab.at[src_v.at[0]], b0, sg0)
        pltpu.async_copy(tab.at[src_v.at[1]], b1, sg1)
        step(0, 0, True, False)
        step(1, 1, True, False)
        step(2, 2, False, False)
        step(3, 3, False, False)

        def body(i, _):
            g = i * 4
            for j in range(4):
                step(g + j, j, False, False)
            return 0

        lax.fori_loop(1, CH // 4 - 1, body, 0)

        g = CH - 4
        step(g, 0, False, False)
        step(g + 1, 1, False, False)
        step(g + 2, 2, False, True)
        step(g + 3, 3, False, True)
        wait_scatter(2)
        wait_scatter(3)
        plsc.subcore_barrier()
        pltpu.sync_copy(acc.at[pl.ds(base, ROWS_PER_TILE)],
                        out_hbm.at[cid, pl.ds(base, ROWS_PER_TILE)])

    return k(u2, src3, dst3)


def _row_spec(d):
    return pl.BlockSpec((RB, d), lambda i: (i, 0))


def _split_spec(w):
    return pl.BlockSpec((2, RB, w), lambda i: (0, i, 0))


def _full_spec(shape):
    nd = len(shape)
    return pl.BlockSpec(shape, lambda i, _n=nd: (0,) * _n)


def _tc_prep(xp, pp, hist):
    def body(x_ref, p_ref, h_ref, u_ref, dis2_ref, sdeg_ref):
        deg = h_ref[0, :, 0:1] + h_ref[1, :, 0:1] + 1.0
        dis = lax.rsqrt(deg)
        u_ref[0, :, :] = x_ref[:, 0:80] * dis
        u_ref[1, :, 0:48] = x_ref[:, 80:128] * dis
        u_ref[1, :, 48:64] = p_ref[...] * dis
        u_ref[1, :, 64:80] = jnp.zeros((RB, 16), _f32)
        dis2_ref[...] = 1.0 / deg
        sdeg_ref[...] = jnp.sqrt(deg)

    return pl.pallas_call(
        body,
        grid=(NBLK,),
        in_specs=[
            _row_spec(128),
            _row_spec(16),
            pl.BlockSpec((2, RB, 16), lambda i: (0, i, 0)),
        ],
        out_specs=[_split_spec(W1H), _row_spec(1), _row_spec(1)],
        out_shape=[
            jax.ShapeDtypeStruct((2, N_PAD, W1H), _f32),
            jax.ShapeDtypeStruct((N_PAD, 1), _f32),
            jax.ShapeDtypeStruct((N_PAD, 1), _f32),
        ],
    )(xp, pp, hist)


def _tc_combine(y2, ukm1, ukm2, dis2, alpha, w):
    has_prev = ukm2 is not None

    def body(*refs):
        if has_prev:
            y_ref, a_ref, b_ref, s_ref, o_ref = refs
        else:
            y_ref, a_ref, s_ref, o_ref = refs
        s = s_ref[...]
        for c in (0, 1):
            o = (-alpha) * s * (y_ref[c] + a_ref[c])
            if has_prev:
                o = o - b_ref[c]
            o_ref[c, :, :] = o

    in_specs = [_split_spec(w), _split_spec(w)]
    args = [y2, ukm1]
    if has_prev:
        in_specs.append(_split_spec(w))
        args.append(ukm2)
    in_specs.append(_row_spec(1))
    args.append(dis2)
    return pl.pallas_call(
        body,
        grid=(NBLK,),
        in_specs=in_specs,
        out_specs=_split_spec(w),
        out_shape=jax.ShapeDtypeStruct((2, N_PAD, w), _f32),
    )(*args)


def _tc_layer1_final(us, sdeg, dis2, wc, b):
    def body(u0, u1, u2, u3, s_ref, s2_ref, w_ref, b_ref, o_ref):
        s = s_ref[...]
        acc = b_ref[...] + jnp.zeros((RB, 128), _f32)
        for k, u in enumerate((u0, u1, u2, u3)):
            acc += jnp.dot(u[0] * s, w_ref[pl.ds(144 * k, 80)],
                           preferred_element_type=_f32)
            acc += jnp.dot(u[1, :, 0:64] * s, w_ref[pl.ds(144 * k + 80, 64)],
                           preferred_element_type=_f32)
        h = jnp.maximum(acc, 0.0)
        dis = jnp.sqrt(s2_ref[...])
        o_ref[0, :, :] = h[:, 0:64] * dis
        o_ref[1, :, :] = h[:, 64:128] * dis

    return pl.pallas_call(
        body,
        grid=(NBLK,),
        in_specs=[_split_spec(W1H)] * 4
        + [_row_spec(1), _row_spec(1), _full_spec((576, 128)),
           _full_spec((1, 128))],
        out_specs=_split_spec(W2H),
        out_shape=jax.ShapeDtypeStruct((2, N_PAD, W2H), _f32),
    )(*us, sdeg, dis2, wc, b)


def _tc_layer2_final(us, sdeg, wc, b, wmu, bmu, wlv, blv):
    def body(u0, u1, u2, u3, s_ref, w_ref, b_ref,
             wmu_ref, bmu_ref, wlv_ref, blv_ref, mu_ref, lv_ref):
        s = s_ref[...]
        acc = b_ref[...] + jnp.zeros((RB, 128), _f32)
        for k, u in enumerate((u0, u1, u2, u3)):
            acc += jnp.dot(u[0] * s, w_ref[pl.ds(128 * k, 64)],
                           preferred_element_type=_f32)
            acc += jnp.dot(u[1] * s, w_ref[pl.ds(128 * k + 64, 64)],
                           preferred_element_type=_f32)
        h = jnp.maximum(acc, 0.0)
        mu_ref[...] = jnp.dot(h, wmu_ref[...],
                              preferred_element_type=_f32) + bmu_ref[...]
        lv_ref[...] = jnp.dot(h, wlv_ref[...],
                              preferred_element_type=_f32) + blv_ref[...]

    return pl.pallas_call(
        body,
        grid=(NBLK,),
        in_specs=[_split_spec(W2H)] * 4
        + [_row_spec(1), _full_spec((512, 128)), _full_spec((1, 128)),
           _full_spec((128, 64)), _full_spec((1, 64)),
           _full_spec((128, 64)), _full_spec((1, 64))],
        out_specs=[_row_spec(64), _row_spec(64)],
        out_shape=[
            jax.ShapeDtypeStruct((N_PAD, 64), _f32),
            jax.ShapeDtypeStruct((N_PAD, 64), _f32),
        ],
    )(*us, sdeg, wc, b, wmu, bmu, wlv, blv)


def kernel(x, edge_index, lap_pe, W1, b1, W2, b2, Wmu, bmu, Wlv, blv):
    pad_idx = N + jnp.arange(E_PAD - E, dtype=jnp.int32) % (N_PAD - N)
    src3 = jnp.concatenate([edge_index[0], pad_idx]).reshape(16, CH, C)
    dst3 = jnp.concatenate([edge_index[1], pad_idx]).reshape(16, CH, C)
    xp = jnp.pad(x, ((0, N_PAD - N), (0, 0)))
    pp = jnp.pad(lap_pe, ((0, N_PAD - N), (0, 0)))

    hist = _sc_degree(src3)
    u0, dis2, sdeg = _tc_prep(xp, pp, hist)

    y = _sc_prop(u0, src3, dst3, W1H)
    u1 = _tc_combine(y, u0, None, dis2, 1.0, W1H)
    y = _sc_prop(u1, src3, dst3, W1H)
    u2 = _tc_combine(y, u1, u0, dis2, 2.0, W1H)
    y = _sc_prop(u2, src3, dst3, W1H)
    u3 = _tc_combine(y, u2, u1, dis2, 2.0, W1H)

    v0 = _tc_layer1_final((u0, u1, u2, u3), sdeg, dis2,
                          W1.reshape(4 * 144, 128), b1.reshape(1, 128))

    y = _sc_prop(v0, src3, dst3, W2H)
    v1 = _tc_combine(y, v0, None, dis2, 1.0, W2H)
    y = _sc_prop(v1, src3, dst3, W2H)
    v2 = _tc_combine(y, v1, v0, dis2, 2.0, W2H)
    y = _sc_prop(v2, src3, dst3, W2H)
    v3 = _tc_combine(y, v2, v1, dis2, 2.0, W2H)

    mu, lv = _tc_layer2_final((v0, v1, v2, v3), sdeg,
                              W2.reshape(4 * 128, 128), b2.reshape(1, 128),
                              Wmu, bmu.reshape(1, 64),
                              Wlv, blv.reshape(1, 64))
    return mu[:N], lv[:N]


# speedup vs baseline: 2.5406x; 1.0501x over previous
import functools

import jax
import jax.numpy as jnp
from jax import lax
from jax.experimental import pallas as pl
from jax.experimental.pallas import tpu as pltpu
from jax.experimental.pallas import tpu_sc as plsc

N = 10000
E = 320000
N_PAD = 10240
C = 112
CH = 180
E_PAD = 16 * CH * C
RB = 256
NBLK = N_PAD // RB
ROWS_PER_TILE = N_PAD // 16
W1H = 80
W2H = 64

_f32 = jnp.float32


def _mesh():
    return plsc.VectorSubcoreMesh(core_axis_name="c", subcore_axis_name="s")


def _sc_params():
    return pltpu.CompilerParams(use_tc_tiling_on_sc=False)


def _fill_const(ref, rows, d, val):
    nv = d // 16

    def row(i, _):
        for j in range(nv):
            ref[i, pl.ds(j * 16, 16)] = jnp.full((16,), val, _f32)
        return 0

    lax.fori_loop(0, rows, row, 0)


def _sc_degree(src3):
    @functools.partial(
        pl.kernel,
        mesh=_mesh(),
        compiler_params=_sc_params(),
        out_type=jax.ShapeDtypeStruct((2, N_PAD, 16), _f32),
        scratch_types=[
            pltpu.VMEM((CH, C), jnp.int32),
            pltpu.VMEM((C, 16), _f32),
            pltpu.VMEM((C, 16), _f32),
            pltpu.VMEM_SHARED((N_PAD, 16), _f32),
        ],
    )
    def k(src_hbm, out_hbm, idx_v, ones_v, zer_v, acc):
        cid = lax.axis_index("c")
        sid = lax.axis_index("s")
        _fill_const(ones_v, C, 16, 1.0)
        _fill_const(zer_v, C, 16, 0.0)
        base = sid * ROWS_PER_TILE
        nfull, rem = divmod(ROWS_PER_TILE, C)
        for j in range(nfull):
            pltpu.sync_copy(zer_v, acc.at[pl.ds(base + j * C, C)])
        if rem:
            pltpu.sync_copy(zer_v.at[pl.ds(0, rem)],
                            acc.at[pl.ds(base + nfull * C, rem)])
        plsc.subcore_barrier()

        pltpu.sync_copy(src_hbm.at[sid], idx_v)
        lo = cid * (CH // 2)
        hi = lo + (CH // 2)

        def body(g, _):
            pltpu.sync_copy(ones_v, acc.at[idx_v.at[g]], add=True)
            return 0

        lax.fori_loop(lo, hi, body, 0)
        plsc.subcore_barrier()
        pltpu.sync_copy(acc.at[pl.ds(base, ROWS_PER_TILE)],
                        out_hbm.at[cid, pl.ds(base, ROWS_PER_TILE)])

    return k(src3)


def _sc_layer(u0, src3, dst3, dis2_1d, w):
    out_t = jax.ShapeDtypeStruct((2, N_PAD, w), _f32)

    @functools.partial(
        pl.kernel,
        mesh=_mesh(),
        compiler_params=_sc_params(),
        out_type=(out_t, out_t, out_t),
        scratch_types=[
            pltpu.VMEM((CH, C), jnp.int32),
            pltpu.VMEM((CH, C), jnp.int32),
            pltpu.VMEM((C, w), _f32),
            pltpu.VMEM((C, w), _f32),
            pltpu.VMEM((C, w), _f32),
            pltpu.VMEM((C, w), _f32),
            pltpu.VMEM((ROWS_PER_TILE,), _f32),
            pltpu.VMEM_SHARED((N_PAD, w), _f32),
            pltpu.SemaphoreType.DMA,
            pltpu.SemaphoreType.DMA,
            pltpu.SemaphoreType.DMA,
            pltpu.SemaphoreType.DMA,
        ],
    )
    def k(u0_hbm, src_hbm, dst_hbm, dis2_hbm, u1_hbm, u2_hbm, u3_hbm,
          src_v, dst_v, b0, b1, b2, b3, d2_v, acc, sg0, sg1, ss0, ss1):
        cid = lax.axis_index("c")
        sid = lax.axis_index("s")
        base = sid * ROWS_PER_TILE
        bufs = (b0, b1, b2, b3)
        sgs = (sg0, sg1)
        sss = (ss0, ss1)
        tabs = (u0_hbm, u1_hbm, u2_hbm, u3_hbm)
        nfull, rem = divmod(ROWS_PER_TILE, C)
        chunks = [(i * C, C) for i in range(nfull)]
        if rem:
            chunks.append((nfull * C, rem))

        pltpu.sync_copy(src_hbm.at[sid], src_v)
        pltpu.sync_copy(dst_hbm.at[sid], dst_v)
        pltpu.sync_copy(dis2_hbm.at[pl.ds(base, ROWS_PER_TILE)], d2_v)

        for kk, alpha in ((1, 1.0), (2, 2.0), (3, 2.0)):
            tab = tabs[kk - 1].at[cid]
            out_hbm = tabs[kk]

            _fill_const(b0, C, w, 0.0)
            for off, n in chunks:
                pltpu.sync_copy(b0.at[pl.ds(0, n)],
                                acc.at[pl.ds(base + off, n)])
            plsc.subcore_barrier()

            def wait_gather(slot):
                pltpu.make_async_copy(tab.at[pl.ds(0, C)], bufs[slot],
                                      sgs[slot % 2]).wait()

            def wait_scatter(slot):
                pltpu.make_async_copy(bufs[slot],
                                      out_hbm.at[cid, pl.ds(0, C)],
                                      sss[slot % 2]).wait()

            def step(g, slot, first, last):
                wait_gather(slot)
                pltpu.async_copy(bufs[slot], acc.at[dst_v.at[g]],
                                 sss[slot % 2], add=True)
                if not first:
                    wait_scatter((slot + 2) % 4)
                if not last:
                    pltpu.async_copy(tab.at[src_v.at[g + 2]],
                                     bufs[(slot + 2) % 4], sgs[slot % 2])

            pltpu.async_copy(tab.at[src_v.at[0]], b0, sg0)
            pltpu.async_copy(tab.at[src_v.at[1]], b1, sg1)
            step(0, 0, True, False)
            step(1, 1, True, False)
            step(2, 2, False, False)
            step(3, 3, False, False)

            def body(i, _):
                g = i * 4
                for j in range(4):
                    step(g + j, j, False, False)
                return 0

            lax.fori_loop(1, CH // 4 - 1, body, 0)

            g = CH - 4
            step(g, 0, False, False)
            step(g + 1, 1, False, False)
            step(g + 2, 2, False, True)
            step(g + 3, 3, False, True)
            wait_scatter(2)
            wait_scatter(3)
            plsc.subcore_barrier()

            for off, n in chunks:
                pltpu.sync_copy(acc.at[pl.ds(base + off, n)],
                                b0.at[pl.ds(0, n)])
                pltpu.sync_copy(tabs[kk - 1].at[cid, pl.ds(base + off, n)],
                                b1.at[pl.ds(0, n)])
                if kk >= 2:
                    pltpu.sync_copy(
                        tabs[kk - 2].at[cid, pl.ds(base + off, n)],
                        b2.at[pl.ds(0, n)])

                def row_fn(i, _, _off=off, _kk=kk, _alpha=alpha):
                    dv = d2_v[pl.ds(_off + (i // 16) * 16, 16)]
                    lane = jnp.full((16,), i % 16, jnp.int32)
                    sv = jnp.take_along_axis(
                        dv, lane, axis=0,
                        mode="promise_in_bounds") * (-_alpha)
                    for j in range(w // 16):
                        sl = pl.ds(j * 16, 16)
                        o = sv * (b0[i, sl] + b1[i, sl])
                        if _kk >= 2:
                            o = o - b2[i, sl]
                        b3[i, sl] = o
                    return 0

                lax.fori_loop(0, n, row_fn, 0)
                pltpu.sync_copy(b3.at[pl.ds(0, n)],
                                out_hbm.at[cid, pl.ds(base + off, n)])
            plsc.subcore_barrier()

    return k(u0, src3, dst3, dis2_1d)


def _row_spec(d):
    return pl.BlockSpec((RB, d), lambda i: (i, 0))


def _split_spec(w):
    return pl.BlockSpec((2, RB, w), lambda i: (0, i, 0))


def _full_spec(shape):
    nd = len(shape)
    return pl.BlockSpec(shape, lambda i, _n=nd: (0,) * _n)


def _tc_prep(xp, pp, hist):
    def body(x_ref, p_ref, h_ref, u_ref, dis2_ref, sdeg_ref):
        deg = h_ref[0, :, 0:1] + h_ref[1, :, 0:1] + 1.0
        dis = lax.rsqrt(deg)
        u_ref[0, :, :] = x_ref[:, 0:80] * dis
        u_ref[1, :, 0:48] = x_ref[:, 80:128] * dis
        u_ref[1, :, 48:64] = p_ref[...] * dis
        u_ref[1, :, 64:80] = jnp.zeros((RB, 16), _f32)
        dis2_ref[...] = 1.0 / deg
        sdeg_ref[...] = jnp.sqrt(deg)

    return pl.pallas_call(
        body,
        grid=(NBLK,),
        in_specs=[
            _row_spec(128),
            _row_spec(16),
            pl.BlockSpec((2, RB, 16), lambda i: (0, i, 0)),
        ],
        out_specs=[_split_spec(W1H), _row_spec(1), _row_spec(1)],
        out_shape=[
            jax.ShapeDtypeStruct((2, N_PAD, W1H), _f32),
            jax.ShapeDtypeStruct((N_PAD, 1), _f32),
            jax.ShapeDtypeStruct((N_PAD, 1), _f32),
        ],
    )(xp, pp, hist)


def _tc_layer1_final(us, sdeg, dis2, wc, b):
    def body(u0, u1, u2, u3, s_ref, s2_ref, w_ref, b_ref, o_ref):
        s = s_ref[...]
        acc = b_ref[...] + jnp.zeros((RB, 128), _f32)
        for k, u in enumerate((u0, u1, u2, u3)):
            acc += jnp.dot(u[0] * s, w_ref[pl.ds(144 * k, 80)],
                           preferred_element_type=_f32)
            acc += jnp.dot(u[1, :, 0:64] * s, w_ref[pl.ds(144 * k + 80, 64)],
                           preferred_element_type=_f32)
        h = jnp.maximum(acc, 0.0)
        dis = jnp.sqrt(s2_ref[...])
        o_ref[0, :, :] = h[:, 0:64] * dis
        o_ref[1, :, :] = h[:, 64:128] * dis

    return pl.pallas_call(
        body,
        grid=(NBLK,),
        in_specs=[_split_spec(W1H)] * 4
        + [_row_spec(1), _row_spec(1), _full_spec((576, 128)),
           _full_spec((1, 128))],
        out_specs=_split_spec(W2H),
        out_shape=jax.ShapeDtypeStruct((2, N_PAD, W2H), _f32),
    )(*us, sdeg, dis2, wc, b)


def _tc_layer2_final(us, sdeg, wc, b, wmu, bmu, wlv, blv):
    def body(u0, u1, u2, u3, s_ref, w_ref, b_ref,
             wmu_ref, bmu_ref, wlv_ref, blv_ref, mu_ref, lv_ref):
        s = s_ref[...]
        acc = b_ref[...] + jnp.zeros((RB, 128), _f32)
        for k, u in enumerate((u0, u1, u2, u3)):
            acc += jnp.dot(u[0] * s, w_ref[pl.ds(128 * k, 64)],
                           preferred_element_type=_f32)
            acc += jnp.dot(u[1] * s, w_ref[pl.ds(128 * k + 64, 64)],
                           preferred_element_type=_f32)
        h = jnp.maximum(acc, 0.0)
        mu_ref[...] = jnp.dot(h, wmu_ref[...],
                              preferred_element_type=_f32) + bmu_ref[...]
        lv_ref[...] = jnp.dot(h, wlv_ref[...],
                              preferred_element_type=_f32) + blv_ref[...]

    return pl.pallas_call(
        body,
        grid=(NBLK,),
        in_specs=[_split_spec(W2H)] * 4
        + [_row_spec(1), _full_spec((512, 128)), _full_spec((1, 128)),
           _full_spec((128, 64)), _full_spec((1, 64)),
           _full_spec((128, 64)), _full_spec((1, 64))],
        out_specs=[_row_spec(64), _row_spec(64)],
        out_shape=[
            jax.ShapeDtypeStruct((N_PAD, 64), _f32),
            jax.ShapeDtypeStruct((N_PAD, 64), _f32),
        ],
    )(*us, sdeg, wc, b, wmu, bmu, wlv, blv)


def kernel(x, edge_index, lap_pe, W1, b1, W2, b2, Wmu, bmu, Wlv, blv):
    pad_idx = N + jnp.arange(E_PAD - E, dtype=jnp.int32) % (N_PAD - N)
    src3 = jnp.concatenate([edge_index[0], pad_idx]).reshape(16, CH, C)
    dst3 = jnp.concatenate([edge_index[1], pad_idx]).reshape(16, CH, C)
    xp = jnp.pad(x, ((0, N_PAD - N), (0, 0)))
    pp = jnp.pad(lap_pe, ((0, N_PAD - N), (0, 0)))

    hist = _sc_degree(src3)
    u0, dis2, sdeg = _tc_prep(xp, pp, hist)
    dis2_1d = dis2.reshape(N_PAD)

    u1, u2, u3 = _sc_layer(u0, src3, dst3, dis2_1d, W1H)

    v0 = _tc_layer1_final((u0, u1, u2, u3), sdeg, dis2,
                          W1.reshape(4 * 144, 128), b1.reshape(1, 128))

    v1, v2, v3 = _sc_layer(v0, src3, dst3, dis2_1d, W2H)

    mu, lv = _tc_layer2_final((v0, v1, v2, v3), sdeg,
                              W2.reshape(4 * 128, 128), b2.reshape(1, 128),
                              Wmu, bmu.reshape(1, 64),
                              Wlv, blv.reshape(1, 64))
    return mu[:N], lv[:N]
